# Initial kernel scaffold; baseline (speedup 1.0000x reference)
import jax
import jax.numpy as jnp
from jax.experimental import pallas as pl


def kernel(byte_seq, emb, Wsp, bsp, Wg, bg, Wr, W1, b1, W2, b2, Wh, bh):
    raise NotImplementedError("write your pallas kernel here")


# speedup vs baseline: 11.3334x; 11.3334x over previous
import jax
import jax.numpy as jnp
from jax import lax
from jax.experimental import pallas as pl
from jax.experimental.pallas import tpu as pltpu

_B, _T, _C, _V, _E, _F = 2, 2048, 1024, 256, 8, 2048
_N = _B * _T
_CAP = int(1.25 * _N * 2 / _E)

_TCE = 512
_TCS = 256
_NCF = 512
_NCH = 2048


def _tables_k(emb_ref, wsp_ref, bsp_ref, wg_ref, bg_ref, tsp_ref, tg_ref):
    emb = emb_ref[...]
    tsp_ref[...] = jnp.dot(emb, wsp_ref[...],
                           preferred_element_type=jnp.float32) + bsp_ref[...]
    tg_ref[...] = jnp.dot(emb, wg_ref[...],
                          preferred_element_type=jnp.float32) + bg_ref[...]


def _embed_k(bs_ref, tsp_ref, tg_ref, a_ref, s_ref):
    iota_v = lax.broadcasted_iota(jnp.int32, (_TCE, _V), 1)
    for b in range(_B):
        tok = bs_ref[...][:, b]
        oh = (tok[:, None] == iota_v).astype(jnp.float32)
        a_ref[:, b, :] = jnp.dot(oh, tsp_ref[...],
                                 preferred_element_type=jnp.float32,
                                 precision=lax.Precision.HIGHEST)
        s_ref[:, b, :] = jax.nn.sigmoid(
            jnp.dot(oh, tg_ref[...], preferred_element_type=jnp.float32,
                    precision=lax.Precision.HIGHEST))


def _scan_k(a_ref, s_ref, xs_ref, fin_ref, st_ref):
    @pl.when(pl.program_id(0) == 0)
    def _init():
        st_ref[...] = jnp.zeros((16, 128), jnp.float32)

    def body(i, st):
        ns = jnp.tanh(a_ref[i] + st)
        xs_ref[i] = s_ref[i] * ns
        return ns

    st = lax.fori_loop(0, _TCS, body, st_ref[...])
    st_ref[...] = st
    fin_ref[...] = st


def _router_k(x_ref, wr_ref, gate_ref, psum_ref, cnt_ref, ent_ref):
    x = x_ref[...]
    logits = jnp.dot(x, wr_ref[...], preferred_element_type=jnp.float32)
    m = jnp.max(logits, axis=1, keepdims=True)
    ex = jnp.exp(logits - m)
    probs = ex / jnp.sum(ex, axis=1, keepdims=True)

    iota8 = lax.broadcasted_iota(jnp.int32, (_N, _E), 1)
    m1 = jnp.max(probs, axis=1, keepdims=True)
    i1 = jnp.min(jnp.where(probs == m1, iota8, _E), axis=1, keepdims=True)
    oh1 = i1 == iota8
    masked = jnp.where(oh1, -1.0, probs)
    m2 = jnp.max(masked, axis=1, keepdims=True)
    i2 = jnp.min(jnp.where(masked == m2, iota8, _E), axis=1, keepdims=True)
    oh2 = i2 == iota8
    denom = m1 + m2 + 1e-9
    gate = jnp.where(oh1, m1 / denom, 0.0) + jnp.where(oh2, m2 / denom, 0.0)

    cnts = jnp.sum((oh1 | oh2).astype(jnp.int32), axis=0, keepdims=True)
    cnt_ref[...] = cnts
    psum = jnp.sum(probs, axis=0, keepdims=True)
    psum_ref[...] = psum
    mp = psum / _N
    ent_ref[...] = jnp.sum(mp * jnp.log(mp + 1e-9), axis=1, keepdims=True)

    ov = jnp.max(cnts)

    @pl.when(ov > _CAP)
    def _overflow():
        bits = lax.bitcast_convert_type(gate, jnp.int32)

        def bis(_, carry):
            lo, hi = carry
            mid = lo + ((hi - lo + 1) >> 1)
            c = jnp.sum((bits >= mid).astype(jnp.int32), axis=0,
                        keepdims=True)
            good = c >= _CAP
            return jnp.where(good, mid, lo), jnp.where(good, hi, mid - 1)

        lo0 = jnp.zeros((1, _E), jnp.int32)
        hi0 = jnp.full((1, _E), 0x3F800000, jnp.int32)
        lo, _ = lax.fori_loop(0, 31, bis, (lo0, hi0))
        gate_ref[...] = jnp.where(bits >= jnp.maximum(lo, 1), gate, 0.0)

    @pl.when(jnp.logical_not(ov > _CAP))
    def _no_overflow():
        gate_ref[...] = gate


def _ffn_k(x_ref, gm_ref, w1_ref, b1_ref, w2_ref, b2_ref, out_ref):
    e = pl.program_id(1)
    x = x_ref[...]
    h = jnp.maximum(
        jnp.dot(x, w1_ref[0], preferred_element_type=jnp.float32)
        + b1_ref[0], 0.0)
    y = jnp.dot(h, w2_ref[0], preferred_element_type=jnp.float32) + b2_ref[0]
    iota8 = lax.broadcasted_iota(jnp.int32, (_NCF, _E), 1)
    ge = jnp.sum(jnp.where(iota8 == e, gm_ref[...], 0.0), axis=1,
                 keepdims=True)

    @pl.when(e == 0)
    def _first():
        out_ref[...] = y * ge

    @pl.when(e > 0)
    def _rest():
        out_ref[...] += y * ge


def _head_k(x_ref, wh_ref, bh_ref, o_ref):
    o_ref[...] = jnp.dot(x_ref[...], wh_ref[...],
                         preferred_element_type=jnp.float32) + bh_ref[...]


def kernel(byte_seq, emb, Wsp, bsp, Wg, bg, Wr, W1, b1, W2, b2, Wh, bh):
    f32 = jnp.float32
    bs_t = byte_seq.astype(jnp.int32).T

    tsp, tg = pl.pallas_call(
        _tables_k,
        out_shape=[jax.ShapeDtypeStruct((_V, _C), f32)] * 2,
    )(emb, Wsp, bsp.reshape(1, _C), Wg, bg.reshape(1, _C))

    a, s = pl.pallas_call(
        _embed_k,
        grid=(_T // _TCE,),
        in_specs=[
            pl.BlockSpec((_TCE, _B), lambda i: (i, 0)),
            pl.BlockSpec((_V, _C), lambda i: (0, 0)),
            pl.BlockSpec((_V, _C), lambda i: (0, 0)),
        ],
        out_specs=[
            pl.BlockSpec((_TCE, _B, _C), lambda i: (i, 0, 0)),
            pl.BlockSpec((_TCE, _B, _C), lambda i: (i, 0, 0)),
        ],
        out_shape=[jax.ShapeDtypeStruct((_T, _B, _C), f32)] * 2,
    )(bs_t, tsp, tg)

    a3 = a.reshape(_T, 16, 128)
    s3 = s.reshape(_T, 16, 128)

    xs, fin = pl.pallas_call(
        _scan_k,
        grid=(_T // _TCS,),
        in_specs=[
            pl.BlockSpec((_TCS, 16, 128), lambda i: (i, 0, 0)),
            pl.BlockSpec((_TCS, 16, 128), lambda i: (i, 0, 0)),
        ],
        out_specs=[
            pl.BlockSpec((_TCS, 16, 128), lambda i: (i, 0, 0)),
            pl.BlockSpec((16, 128), lambda i: (0, 0)),
        ],
        out_shape=[
            jax.ShapeDtypeStruct((_T, 16, 128), f32),
            jax.ShapeDtypeStruct((16, 128), f32),
        ],
        scratch_shapes=[pltpu.VMEM((16, 128), f32)],
    )(a3, s3)

    x_flat = xs.reshape(_T, _B, _C).transpose(1, 0, 2).reshape(_N, _C)

    gate, psum, cnt, ent = pl.pallas_call(
        _router_k,
        out_shape=[
            jax.ShapeDtypeStruct((_N, _E), f32),
            jax.ShapeDtypeStruct((1, _E), f32),
            jax.ShapeDtypeStruct((1, _E), jnp.int32),
            jax.ShapeDtypeStruct((1, 1), f32),
        ],
    )(x_flat, Wr)

    moe = pl.pallas_call(
        _ffn_k,
        grid=(_N // _NCF, _E),
        in_specs=[
            pl.BlockSpec((_NCF, _C), lambda i, e: (i, 0)),
            pl.BlockSpec((_NCF, _E), lambda i, e: (i, 0)),
            pl.BlockSpec((1, _C, _F), lambda i, e: (e, 0, 0)),
            pl.BlockSpec((1, 1, _F), lambda i, e: (e, 0, 0)),
            pl.BlockSpec((1, _F, _C), lambda i, e: (e, 0, 0)),
            pl.BlockSpec((1, 1, _C), lambda i, e: (e, 0, 0)),
        ],
        out_specs=pl.BlockSpec((_NCF, _C), lambda i, e: (i, 0)),
        out_shape=jax.ShapeDtypeStruct((_N, _C), f32),
    )(x_flat, gate, W1, b1.reshape(_E, 1, _F), W2, b2.reshape(_E, 1, _C))

    logits = pl.pallas_call(
        _head_k,
        grid=(_N // _NCH,),
        in_specs=[
            pl.BlockSpec((_NCH, _C), lambda i: (i, 0)),
            pl.BlockSpec((_C, _V), lambda i: (0, 0)),
            pl.BlockSpec((1, _V), lambda i: (0, 0)),
        ],
        out_specs=pl.BlockSpec((_NCH, _V), lambda i: (i, 0)),
        out_shape=jax.ShapeDtypeStruct((_N, _V), f32),
    )(moe, Wh, bh.reshape(1, _V))

    return (
        logits.reshape(_B, _T, _V),
        fin.reshape(_B, _C),
        ent.reshape(()),
        cnt.reshape(_E),
    )


# speedup vs baseline: 12.1606x; 1.0730x over previous
import jax
import jax.numpy as jnp
from jax import lax
from jax.experimental import pallas as pl
from jax.experimental.pallas import tpu as pltpu

_B, _T, _C, _V, _E, _F = 2, 2048, 1024, 256, 8, 2048
_N = _B * _T
_CAP = int(1.25 * _N * 2 / _E)

_TCE = 512
_TCS = 256
_NCF = 1024
_FC = 1024
_NCH = 2048


def _tables_k(emb_ref, wsp_ref, bsp_ref, wg_ref, bg_ref, tsp_ref, tg_ref):
    emb = emb_ref[...]
    tsp_ref[...] = jnp.dot(emb, wsp_ref[...],
                           preferred_element_type=jnp.float32) + bsp_ref[...]
    tg_ref[...] = jnp.dot(emb, wg_ref[...],
                          preferred_element_type=jnp.float32) + bg_ref[...]


def _embed_k(bs_ref, tsp_ref, tg_ref, a_ref, s_ref):
    iota_v = lax.broadcasted_iota(jnp.int32, (_TCE, _V), 1)
    for b in range(_B):
        tok = bs_ref[...][:, b]
        oh = (tok[:, None] == iota_v).astype(jnp.float32)
        a_ref[:, b, :] = jnp.dot(oh, tsp_ref[...],
                                 preferred_element_type=jnp.float32,
                                 precision=lax.Precision.HIGHEST)
        s_ref[:, b, :] = jax.nn.sigmoid(
            jnp.dot(oh, tg_ref[...], preferred_element_type=jnp.float32,
                    precision=lax.Precision.HIGHEST))


def _scan_k(a_ref, s_ref, xs_ref, fin_ref, st_ref):
    @pl.when(pl.program_id(0) == 0)
    def _init():
        st_ref[...] = jnp.zeros((16, 128), jnp.float32)

    def body(i, st):
        ns = jnp.tanh(a_ref[i] + st)
        xs_ref[i] = s_ref[i] * ns
        return ns

    st = lax.fori_loop(0, _TCS, body, st_ref[...])
    st_ref[...] = st
    fin_ref[...] = st


def _router_k(x_ref, wr_ref, gate_ref, psum_ref, cnt_ref, ent_ref):
    x = x_ref[...]
    logits = jnp.dot(x, wr_ref[...], preferred_element_type=jnp.float32)
    m = jnp.max(logits, axis=1, keepdims=True)
    ex = jnp.exp(logits - m)
    probs = ex / jnp.sum(ex, axis=1, keepdims=True)

    iota8 = lax.broadcasted_iota(jnp.int32, (_N, _E), 1)
    m1 = jnp.max(probs, axis=1, keepdims=True)
    i1 = jnp.min(jnp.where(probs == m1, iota8, _E), axis=1, keepdims=True)
    oh1 = i1 == iota8
    masked = jnp.where(oh1, -1.0, probs)
    m2 = jnp.max(masked, axis=1, keepdims=True)
    i2 = jnp.min(jnp.where(masked == m2, iota8, _E), axis=1, keepdims=True)
    oh2 = i2 == iota8
    denom = m1 + m2 + 1e-9
    gate = jnp.where(oh1, m1 / denom, 0.0) + jnp.where(oh2, m2 / denom, 0.0)

    cnts = jnp.sum((oh1 | oh2).astype(jnp.int32), axis=0, keepdims=True)
    cnt_ref[...] = cnts
    psum = jnp.sum(probs, axis=0, keepdims=True)
    psum_ref[...] = psum
    mp = psum / _N
    ent_ref[...] = jnp.sum(mp * jnp.log(mp + 1e-9), axis=1, keepdims=True)

    ov = jnp.max(cnts)

    @pl.when(ov > _CAP)
    def _overflow():
        bits = lax.bitcast_convert_type(gate, jnp.int32)

        def bis(_, carry):
            lo, hi = carry
            mid = lo + ((hi - lo + 1) >> 1)
            c = jnp.sum((bits >= mid).astype(jnp.int32), axis=0,
                        keepdims=True)
            good = c >= _CAP
            return jnp.where(good, mid, lo), jnp.where(good, hi, mid - 1)

        lo0 = jnp.zeros((1, _E), jnp.int32)
        hi0 = jnp.full((1, _E), 0x3F800000, jnp.int32)
        lo, _ = lax.fori_loop(0, 31, bis, (lo0, hi0))
        gate_ref[...] = jnp.where(bits >= jnp.maximum(lo, 1), gate, 0.0)

    @pl.when(jnp.logical_not(ov > _CAP))
    def _no_overflow():
        gate_ref[...] = gate


def _ffn_k(x_ref, gm_ref, w1_ref, b1_ref, w2_ref, b2_ref, out_ref):
    e = pl.program_id(0)
    f = pl.program_id(1)

    @pl.when(jnp.logical_and(e == 0, f == 0))
    def _zero():
        out_ref[...] = jnp.zeros_like(out_ref)

    iota8 = lax.broadcasted_iota(jnp.int32, (_NCF, _E), 1)
    for r in range(_N // _NCF):
        sl = pl.ds(r * _NCF, _NCF)
        x = x_ref[sl, :]
        h = jnp.maximum(
            jnp.dot(x, w1_ref[0], preferred_element_type=jnp.float32)
            + b1_ref[0], 0.0)
        y = jnp.dot(h, w2_ref[0], preferred_element_type=jnp.float32)
        ge = jnp.sum(jnp.where(iota8 == e, gm_ref[sl, :], 0.0), axis=1,
                     keepdims=True)
        y = jnp.where(f == 0, y + b2_ref[0], y)
        out_ref[sl, :] += y * ge


def _head_k(x_ref, wh_ref, bh_ref, o_ref):
    o_ref[...] = jnp.dot(x_ref[...], wh_ref[...],
                         preferred_element_type=jnp.float32) + bh_ref[...]


def kernel(byte_seq, emb, Wsp, bsp, Wg, bg, Wr, W1, b1, W2, b2, Wh, bh):
    f32 = jnp.float32
    bs_t = byte_seq.astype(jnp.int32).T

    tsp, tg = pl.pallas_call(
        _tables_k,
        out_shape=[jax.ShapeDtypeStruct((_V, _C), f32)] * 2,
    )(emb, Wsp, bsp.reshape(1, _C), Wg, bg.reshape(1, _C))

    a, s = pl.pallas_call(
        _embed_k,
        grid=(_T // _TCE,),
        in_specs=[
            pl.BlockSpec((_TCE, _B), lambda i: (i, 0)),
            pl.BlockSpec((_V, _C), lambda i: (0, 0)),
            pl.BlockSpec((_V, _C), lambda i: (0, 0)),
        ],
        out_specs=[
            pl.BlockSpec((_TCE, _B, _C), lambda i: (i, 0, 0)),
            pl.BlockSpec((_TCE, _B, _C), lambda i: (i, 0, 0)),
        ],
        out_shape=[jax.ShapeDtypeStruct((_T, _B, _C), f32)] * 2,
    )(bs_t, tsp, tg)

    a3 = a.reshape(_T, 16, 128)
    s3 = s.reshape(_T, 16, 128)

    xs, fin = pl.pallas_call(
        _scan_k,
        grid=(_T // _TCS,),
        in_specs=[
            pl.BlockSpec((_TCS, 16, 128), lambda i: (i, 0, 0)),
            pl.BlockSpec((_TCS, 16, 128), lambda i: (i, 0, 0)),
        ],
        out_specs=[
            pl.BlockSpec((_TCS, 16, 128), lambda i: (i, 0, 0)),
            pl.BlockSpec((16, 128), lambda i: (0, 0)),
        ],
        out_shape=[
            jax.ShapeDtypeStruct((_T, 16, 128), f32),
            jax.ShapeDtypeStruct((16, 128), f32),
        ],
        scratch_shapes=[pltpu.VMEM((16, 128), f32)],
    )(a3, s3)

    x_flat = xs.reshape(_T, _B, _C).transpose(1, 0, 2).reshape(_N, _C)

    gate, psum, cnt, ent = pl.pallas_call(
        _router_k,
        out_shape=[
            jax.ShapeDtypeStruct((_N, _E), f32),
            jax.ShapeDtypeStruct((1, _E), f32),
            jax.ShapeDtypeStruct((1, _E), jnp.int32),
            jax.ShapeDtypeStruct((1, 1), f32),
        ],
    )(x_flat, Wr)

    moe = pl.pallas_call(
        _ffn_k,
        grid=(_E, _F // _FC),
        in_specs=[
            pl.BlockSpec((_N, _C), lambda e, f: (0, 0)),
            pl.BlockSpec((_N, _E), lambda e, f: (0, 0)),
            pl.BlockSpec((1, _C, _FC), lambda e, f: (e, 0, f)),
            pl.BlockSpec((1, 1, _FC), lambda e, f: (e, 0, f)),
            pl.BlockSpec((1, _FC, _C), lambda e, f: (e, f, 0)),
            pl.BlockSpec((1, 1, _C), lambda e, f: (e, 0, 0)),
        ],
        out_specs=pl.BlockSpec((_N, _C), lambda e, f: (0, 0)),
        out_shape=jax.ShapeDtypeStruct((_N, _C), f32),
    )(x_flat, gate, W1, b1.reshape(_E, 1, _F), W2, b2.reshape(_E, 1, _C))

    logits = pl.pallas_call(
        _head_k,
        grid=(_N // _NCH,),
        in_specs=[
            pl.BlockSpec((_NCH, _C), lambda i: (i, 0)),
            pl.BlockSpec((_C, _V), lambda i: (0, 0)),
            pl.BlockSpec((1, _V), lambda i: (0, 0)),
        ],
        out_specs=pl.BlockSpec((_NCH, _V), lambda i: (i, 0)),
        out_shape=jax.ShapeDtypeStruct((_N, _V), f32),
    )(moe, Wh, bh.reshape(1, _V))

    return (
        logits.reshape(_B, _T, _V),
        fin.reshape(_B, _C),
        ent.reshape(()),
        cnt.reshape(_E),
    )


# speedup vs baseline: 12.2189x; 1.0048x over previous
import jax
import jax.numpy as jnp
from jax import lax
from jax.experimental import pallas as pl
from jax.experimental.pallas import tpu as pltpu

_B, _T, _C, _V, _E, _F = 2, 2048, 1024, 256, 8, 2048
_N = _B * _T
_CAP = int(1.25 * _N * 2 / _E)

_TCE = 512
_TCS = 256
_NCF = 1024
_FC = 1024
_NCH = 2048


def _tables_k(emb_ref, wsp_ref, bsp_ref, wg_ref, bg_ref, tsp_ref, tg_ref):
    emb = emb_ref[...]
    tsp_ref[...] = jnp.dot(emb, wsp_ref[...],
                           preferred_element_type=jnp.float32) + bsp_ref[...]
    tg_ref[...] = jnp.dot(emb, wg_ref[...],
                          preferred_element_type=jnp.float32) + bg_ref[...]


def _embed_k(bs_ref, tsp_ref, tg_ref, a_ref, s_ref):
    iota_v = lax.broadcasted_iota(jnp.int32, (_TCE, _V), 1)
    for b in range(_B):
        tok = bs_ref[...][:, b]
        oh = (tok[:, None] == iota_v).astype(jnp.float32)
        a_ref[:, b, :] = jnp.dot(oh, tsp_ref[...],
                                 preferred_element_type=jnp.float32,
                                 precision=lax.Precision.HIGHEST)
        s_ref[:, b, :] = jax.nn.sigmoid(
            jnp.dot(oh, tg_ref[...], preferred_element_type=jnp.float32,
                    precision=lax.Precision.HIGHEST))


def _scan_k(a_ref, s_ref, xs_ref, fin_ref, st_ref):
    @pl.when(pl.program_id(0) == 0)
    def _init():
        st_ref[...] = jnp.zeros((16, 128), jnp.float32)

    def body(i, st):
        ns = jnp.tanh(a_ref[i] + st)
        xs_ref[i] = s_ref[i] * ns
        return ns

    st = lax.fori_loop(0, _TCS, body, st_ref[...], unroll=8)
    st_ref[...] = st
    fin_ref[...] = st


def _router_k(x_ref, wr_ref, gate_ref, psum_ref, cnt_ref, ent_ref):
    x = x_ref[...]
    logits = jnp.dot(x, wr_ref[...], preferred_element_type=jnp.float32)
    m = jnp.max(logits, axis=1, keepdims=True)
    ex = jnp.exp(logits - m)
    probs = ex / jnp.sum(ex, axis=1, keepdims=True)

    iota8 = lax.broadcasted_iota(jnp.int32, (_N, _E), 1)
    m1 = jnp.max(probs, axis=1, keepdims=True)
    i1 = jnp.min(jnp.where(probs == m1, iota8, _E), axis=1, keepdims=True)
    oh1 = i1 == iota8
    masked = jnp.where(oh1, -1.0, probs)
    m2 = jnp.max(masked, axis=1, keepdims=True)
    i2 = jnp.min(jnp.where(masked == m2, iota8, _E), axis=1, keepdims=True)
    oh2 = i2 == iota8
    denom = m1 + m2 + 1e-9
    gate = jnp.where(oh1, m1 / denom, 0.0) + jnp.where(oh2, m2 / denom, 0.0)

    cnts = jnp.sum((oh1 | oh2).astype(jnp.int32), axis=0, keepdims=True)
    cnt_ref[...] = cnts
    psum = jnp.sum(probs, axis=0, keepdims=True)
    psum_ref[...] = psum
    mp = psum / _N
    ent_ref[...] = jnp.sum(mp * jnp.log(mp + 1e-9), axis=1, keepdims=True)

    ov = jnp.max(cnts)

    @pl.when(ov > _CAP)
    def _overflow():
        bits = lax.bitcast_convert_type(gate, jnp.int32)

        def bis(_, carry):
            lo, hi = carry
            mid = lo + ((hi - lo + 1) >> 1)
            c = jnp.sum((bits >= mid).astype(jnp.int32), axis=0,
                        keepdims=True)
            good = c >= _CAP
            return jnp.where(good, mid, lo), jnp.where(good, hi, mid - 1)

        lo0 = jnp.zeros((1, _E), jnp.int32)
        hi0 = jnp.full((1, _E), 0x3F800000, jnp.int32)
        lo, _ = lax.fori_loop(0, 31, bis, (lo0, hi0))
        gate_ref[...] = jnp.where(bits >= jnp.maximum(lo, 1), gate, 0.0)

    @pl.when(jnp.logical_not(ov > _CAP))
    def _no_overflow():
        gate_ref[...] = gate


def _ffn_k(x_ref, gm_ref, w1_ref, b1_ref, w2_ref, b2_ref, out_ref):
    e = pl.program_id(0)
    f = pl.program_id(1)

    @pl.when(jnp.logical_and(e == 0, f == 0))
    def _zero():
        out_ref[...] = jnp.zeros_like(out_ref)

    iota8 = lax.broadcasted_iota(jnp.int32, (_NCF, _E), 1)
    for r in range(_N // _NCF):
        sl = pl.ds(r * _NCF, _NCF)
        x = x_ref[sl, :]
        h = jnp.maximum(
            jnp.dot(x, w1_ref[0], preferred_element_type=jnp.float32)
            + b1_ref[0], 0.0)
        y = jnp.dot(h, w2_ref[0], preferred_element_type=jnp.float32)
        ge = jnp.sum(jnp.where(iota8 == e, gm_ref[sl, :], 0.0), axis=1,
                     keepdims=True)
        y = jnp.where(f == 0, y + b2_ref[0], y)
        out_ref[sl, :] += y * ge


def _head_k(x_ref, wh_ref, bh_ref, o_ref):
    o_ref[...] = jnp.dot(x_ref[...], wh_ref[...],
                         preferred_element_type=jnp.float32) + bh_ref[...]


def kernel(byte_seq, emb, Wsp, bsp, Wg, bg, Wr, W1, b1, W2, b2, Wh, bh):
    f32 = jnp.float32
    bs_t = byte_seq.astype(jnp.int32).T

    tsp, tg = pl.pallas_call(
        _tables_k,
        out_shape=[jax.ShapeDtypeStruct((_V, _C), f32)] * 2,
    )(emb, Wsp, bsp.reshape(1, _C), Wg, bg.reshape(1, _C))

    a, s = pl.pallas_call(
        _embed_k,
        grid=(_T // _TCE,),
        in_specs=[
            pl.BlockSpec((_TCE, _B), lambda i: (i, 0)),
            pl.BlockSpec((_V, _C), lambda i: (0, 0)),
            pl.BlockSpec((_V, _C), lambda i: (0, 0)),
        ],
        out_specs=[
            pl.BlockSpec((_TCE, _B, _C), lambda i: (i, 0, 0)),
            pl.BlockSpec((_TCE, _B, _C), lambda i: (i, 0, 0)),
        ],
        out_shape=[jax.ShapeDtypeStruct((_T, _B, _C), f32)] * 2,
    )(bs_t, tsp, tg)

    a3 = a.reshape(_T, 16, 128)
    s3 = s.reshape(_T, 16, 128)

    xs, fin = pl.pallas_call(
        _scan_k,
        grid=(_T // _TCS,),
        in_specs=[
            pl.BlockSpec((_TCS, 16, 128), lambda i: (i, 0, 0)),
            pl.BlockSpec((_TCS, 16, 128), lambda i: (i, 0, 0)),
        ],
        out_specs=[
            pl.BlockSpec((_TCS, 16, 128), lambda i: (i, 0, 0)),
            pl.BlockSpec((16, 128), lambda i: (0, 0)),
        ],
        out_shape=[
            jax.ShapeDtypeStruct((_T, 16, 128), f32),
            jax.ShapeDtypeStruct((16, 128), f32),
        ],
        scratch_shapes=[pltpu.VMEM((16, 128), f32)],
    )(a3, s3)

    x_flat = xs.reshape(_T, _B, _C).transpose(1, 0, 2).reshape(_N, _C)

    gate, psum, cnt, ent = pl.pallas_call(
        _router_k,
        out_shape=[
            jax.ShapeDtypeStruct((_N, _E), f32),
            jax.ShapeDtypeStruct((1, _E), f32),
            jax.ShapeDtypeStruct((1, _E), jnp.int32),
            jax.ShapeDtypeStruct((1, 1), f32),
        ],
    )(x_flat, Wr)

    moe = pl.pallas_call(
        _ffn_k,
        grid=(_E, _F // _FC),
        in_specs=[
            pl.BlockSpec((_N, _C), lambda e, f: (0, 0)),
            pl.BlockSpec((_N, _E), lambda e, f: (0, 0)),
            pl.BlockSpec((1, _C, _FC), lambda e, f: (e, 0, f)),
            pl.BlockSpec((1, 1, _FC), lambda e, f: (e, 0, f)),
            pl.BlockSpec((1, _FC, _C), lambda e, f: (e, f, 0)),
            pl.BlockSpec((1, 1, _C), lambda e, f: (e, 0, 0)),
        ],
        out_specs=pl.BlockSpec((_N, _C), lambda e, f: (0, 0)),
        out_shape=jax.ShapeDtypeStruct((_N, _C), f32),
    )(x_flat, gate, W1, b1.reshape(_E, 1, _F), W2, b2.reshape(_E, 1, _C))

    logits = pl.pallas_call(
        _head_k,
        grid=(_N // _NCH,),
        in_specs=[
            pl.BlockSpec((_NCH, _C), lambda i: (i, 0)),
            pl.BlockSpec((_C, _V), lambda i: (0, 0)),
            pl.BlockSpec((1, _V), lambda i: (0, 0)),
        ],
        out_specs=pl.BlockSpec((_NCH, _V), lambda i: (i, 0)),
        out_shape=jax.ShapeDtypeStruct((_N, _V), f32),
    )(moe, Wh, bh.reshape(1, _V))

    return (
        logits.reshape(_B, _T, _V),
        fin.reshape(_B, _C),
        ent.reshape(()),
        cnt.reshape(_E),
    )


# speedup vs baseline: 12.7988x; 1.0475x over previous
import jax
import jax.numpy as jnp
from jax import lax
from jax.experimental import pallas as pl
from jax.experimental.pallas import tpu as pltpu

_B, _T, _C, _V, _E, _F = 2, 2048, 1024, 256, 8, 2048
_N = _B * _T
_CAP = int(1.25 * _N * 2 / _E)

_TCE = 512
_TCS = 256
_NCF = 1024
_FC = 1024
_NCH = 2048


def _tables_k(emb_ref, wsp_ref, bsp_ref, wg_ref, bg_ref, tsp_ref, tg_ref):
    emb = emb_ref[...]
    tsp_ref[...] = jnp.dot(emb, wsp_ref[...],
                           preferred_element_type=jnp.float32) + bsp_ref[...]
    tg_ref[...] = jnp.dot(emb, wg_ref[...],
                          preferred_element_type=jnp.float32) + bg_ref[...]


def _embed_k(bs_ref, tsp_ref, tg_ref, a_ref, s_ref):
    iota_v = lax.broadcasted_iota(jnp.int32, (_TCE, _V), 1)
    for b in range(_B):
        tok = bs_ref[...][:, b]
        oh = (tok[:, None] == iota_v).astype(jnp.float32)
        a_ref[:, b, :] = jnp.dot(oh, tsp_ref[...],
                                 preferred_element_type=jnp.float32,
                                 precision=lax.Precision.HIGHEST)
        s_ref[:, b, :] = jax.nn.sigmoid(
            jnp.dot(oh, tg_ref[...], preferred_element_type=jnp.float32,
                    precision=lax.Precision.HIGHEST))


def _scan_k(a_ref, s_ref, xs_ref, fin_ref, st_ref):
    @pl.when(pl.program_id(0) == 0)
    def _init():
        st_ref[...] = jnp.zeros((16, 128), jnp.float32)

    def body(i, st):
        ns = jnp.tanh(a_ref[i] + st)
        xs_ref[i] = s_ref[i] * ns
        return ns

    st = lax.fori_loop(0, _TCS, body, st_ref[...], unroll=8)
    st_ref[...] = st
    fin_ref[...] = st


def _router_k(x_ref, wr_ref, gate_ref, psum_ref, cnt_ref, ent_ref):
    x = x_ref[...]
    logits = jnp.dot(x, wr_ref[...], preferred_element_type=jnp.float32)
    m = jnp.max(logits, axis=1, keepdims=True)
    ex = jnp.exp(logits - m)
    probs = ex / jnp.sum(ex, axis=1, keepdims=True)

    iota8 = lax.broadcasted_iota(jnp.int32, (_N, _E), 1)
    m1 = jnp.max(probs, axis=1, keepdims=True)
    i1 = jnp.min(jnp.where(probs == m1, iota8, _E), axis=1, keepdims=True)
    oh1 = i1 == iota8
    masked = jnp.where(oh1, -1.0, probs)
    m2 = jnp.max(masked, axis=1, keepdims=True)
    i2 = jnp.min(jnp.where(masked == m2, iota8, _E), axis=1, keepdims=True)
    oh2 = i2 == iota8
    denom = m1 + m2 + 1e-9
    gate = jnp.where(oh1, m1 / denom, 0.0) + jnp.where(oh2, m2 / denom, 0.0)

    cnts = jnp.sum((oh1 | oh2).astype(jnp.int32), axis=0, keepdims=True)
    cnt_ref[...] = cnts
    psum = jnp.sum(probs, axis=0, keepdims=True)
    psum_ref[...] = psum
    mp = psum / _N
    ent_ref[...] = jnp.sum(mp * jnp.log(mp + 1e-9), axis=1, keepdims=True)

    ov = jnp.max(cnts)

    @pl.when(ov > _CAP)
    def _overflow():
        bits = lax.bitcast_convert_type(gate, jnp.int32)

        def bis(_, carry):
            lo, hi = carry
            mid = lo + ((hi - lo + 1) >> 1)
            c = jnp.sum((bits >= mid).astype(jnp.int32), axis=0,
                        keepdims=True)
            good = c >= _CAP
            return jnp.where(good, mid, lo), jnp.where(good, hi, mid - 1)

        lo0 = jnp.zeros((1, _E), jnp.int32)
        hi0 = jnp.full((1, _E), 0x3F800000, jnp.int32)
        lo, _ = lax.fori_loop(0, 31, bis, (lo0, hi0))
        gate_ref[...] = jnp.where(bits >= jnp.maximum(lo, 1), gate, 0.0)

    @pl.when(jnp.logical_not(ov > _CAP))
    def _no_overflow():
        gate_ref[...] = gate


def _ffn_k(x_ref, gm_ref, w1_ref, b1_ref, w2_ref, b2_ref, out_ref):
    e = pl.program_id(0)
    f = pl.program_id(1)

    @pl.when(jnp.logical_and(e == 0, f == 0))
    def _zero():
        out_ref[...] = jnp.zeros_like(out_ref)

    iota8 = lax.broadcasted_iota(jnp.int32, (_NCF, _E), 1)
    for r in range(_N // _NCF):
        sl = pl.ds(r * _NCF, _NCF)
        x = x_ref[sl, :]
        h = jnp.maximum(
            jnp.dot(x, w1_ref[0], preferred_element_type=jnp.float32)
            + b1_ref[0], 0.0)
        y = jnp.dot(h, w2_ref[0], preferred_element_type=jnp.float32)
        ge = jnp.sum(jnp.where(iota8 == e, gm_ref[sl, :], 0.0), axis=1,
                     keepdims=True)
        y = jnp.where(f == 0, y + b2_ref[0], y)
        out_ref[sl, :] += y * ge


def _head_k(x_ref, wh_ref, bh_ref, o_ref):
    o_ref[...] = jnp.dot(x_ref[...], wh_ref[...],
                         preferred_element_type=jnp.float32) + bh_ref[...]


def kernel(byte_seq, emb, Wsp, bsp, Wg, bg, Wr, W1, b1, W2, b2, Wh, bh):
    f32 = jnp.float32
    bs_t = byte_seq.astype(jnp.int32).T

    tsp, tg = pl.pallas_call(
        _tables_k,
        out_shape=[jax.ShapeDtypeStruct((_V, _C), f32)] * 2,
    )(emb, Wsp, bsp.reshape(1, _C), Wg, bg.reshape(1, _C))

    a, s = pl.pallas_call(
        _embed_k,
        grid=(_T // _TCE,),
        in_specs=[
            pl.BlockSpec((_TCE, _B), lambda i: (i, 0)),
            pl.BlockSpec((_V, _C), lambda i: (0, 0)),
            pl.BlockSpec((_V, _C), lambda i: (0, 0)),
        ],
        out_specs=[
            pl.BlockSpec((_TCE, _B, _C), lambda i: (i, 0, 0)),
            pl.BlockSpec((_TCE, _B, _C), lambda i: (i, 0, 0)),
        ],
        out_shape=[jax.ShapeDtypeStruct((_T, _B, _C), f32)] * 2,
    )(bs_t, tsp, tg)

    a3 = a.reshape(_T, 16, 128)
    s3 = s.reshape(_T, 16, 128)

    xs, fin = pl.pallas_call(
        _scan_k,
        grid=(_T // _TCS,),
        in_specs=[
            pl.BlockSpec((_TCS, 16, 128), lambda i: (i, 0, 0)),
            pl.BlockSpec((_TCS, 16, 128), lambda i: (i, 0, 0)),
        ],
        out_specs=[
            pl.BlockSpec((_TCS, 16, 128), lambda i: (i, 0, 0)),
            pl.BlockSpec((16, 128), lambda i: (0, 0)),
        ],
        out_shape=[
            jax.ShapeDtypeStruct((_T, 16, 128), f32),
            jax.ShapeDtypeStruct((16, 128), f32),
        ],
        scratch_shapes=[pltpu.VMEM((16, 128), f32)],
    )(a3, s3)

    x_flat = xs.reshape(_N, _C)

    gate, psum, cnt, ent = pl.pallas_call(
        _router_k,
        out_shape=[
            jax.ShapeDtypeStruct((_N, _E), f32),
            jax.ShapeDtypeStruct((1, _E), f32),
            jax.ShapeDtypeStruct((1, _E), jnp.int32),
            jax.ShapeDtypeStruct((1, 1), f32),
        ],
    )(x_flat, Wr)

    moe = pl.pallas_call(
        _ffn_k,
        grid=(_E, _F // _FC),
        in_specs=[
            pl.BlockSpec((_N, _C), lambda e, f: (0, 0)),
            pl.BlockSpec((_N, _E), lambda e, f: (0, 0)),
            pl.BlockSpec((1, _C, _FC), lambda e, f: (e, 0, f)),
            pl.BlockSpec((1, 1, _FC), lambda e, f: (e, 0, f)),
            pl.BlockSpec((1, _FC, _C), lambda e, f: (e, f, 0)),
            pl.BlockSpec((1, 1, _C), lambda e, f: (e, 0, 0)),
        ],
        out_specs=pl.BlockSpec((_N, _C), lambda e, f: (0, 0)),
        out_shape=jax.ShapeDtypeStruct((_N, _C), f32),
    )(x_flat, gate, W1, b1.reshape(_E, 1, _F), W2, b2.reshape(_E, 1, _C))

    logits = pl.pallas_call(
        _head_k,
        grid=(_N // _NCH,),
        in_specs=[
            pl.BlockSpec((_NCH, _C), lambda i: (i, 0)),
            pl.BlockSpec((_C, _V), lambda i: (0, 0)),
            pl.BlockSpec((1, _V), lambda i: (0, 0)),
        ],
        out_specs=pl.BlockSpec((_NCH, _V), lambda i: (i, 0)),
        out_shape=jax.ShapeDtypeStruct((_N, _V), f32),
    )(moe, Wh, bh.reshape(1, _V))

    return (
        logits.reshape(_T, _B, _V).transpose(1, 0, 2),
        fin.reshape(_B, _C),
        ent.reshape(()),
        cnt.reshape(_E),
    )
